# Initial kernel scaffold; baseline (speedup 1.0000x reference)
import jax
import jax.numpy as jnp
from jax.experimental import pallas as pl


def kernel(x, edge_attr, params, edge_index, batch):
    raise NotImplementedError("write your pallas kernel here")


# speedup vs baseline: 1.0293x; 1.0293x over previous
import jax
import jax.numpy as jnp
from jax.experimental import pallas as pl

_N = 100000
_B = 512
_H = 32


def _boxcox(v):
    return (jnp.sqrt(1.0 + v) - 1.0) * 2.0


def _id_body(a_ref, o_ref):
    o_ref[:] = a_ref[:]


def kernel(x, edge_attr, params, edge_index, batch):
    src = edge_index[0]
    dst = edge_index[1]
    g = _boxcox(edge_attr[:, 0:1])
    c = _boxcox(edge_attr[:, 1:2])
    l = _boxcox(edge_attr[:, 2:3])
    h = x
    for i, lyr in enumerate(params['layers']):
        h_in = h
        xw = h @ lyr['lin_node'].T
        x_j = jnp.take(xw, src, axis=0)

        def phi(p):
            return jnp.maximum(x_j @ p['w1'].T + p['b1'], 0.0) @ p['w2'].T

        msg = g * phi(lyr['phi_G']) + c * phi(lyr['phi_C']) + l * phi(lyr['phi_L'])
        agg = jnp.zeros_like(xw).at[dst].add(msg)
        out = agg + lyr['bias']
        mean = out.mean(axis=-1, keepdims=True)
        var = ((out - mean) ** 2).mean(axis=-1, keepdims=True)
        out = (out - mean) / jnp.sqrt(var + 1e-5) * lyr['ln_g'] + lyr['ln_b']
        if i == 0:
            res = h_in @ params['proj']['w'].T + params['proj']['b']
        else:
            res = h_in
        h = jnp.maximum(out, 0.0) + res

    node_types = jnp.argmax(x, axis=-1)
    nid = jnp.arange(_N, dtype=jnp.int32)

    def term(t):
        key = jnp.where(node_types == t, batch, _B).astype(jnp.int32)
        win = jnp.full((_B + 1,), -1, jnp.int32).at[key].max(nid)[:_B]
        return jnp.where(win[:, None] >= 0, h[jnp.maximum(win, 0)], 0.0)

    vin = term(1)
    vout = term(2)
    gnd = term(0)
    cat = jnp.concatenate([vin, vout, gnd], axis=-1)
    hd = params['heads']
    mu = jnp.concatenate([
        cat @ hd['mu_topo_w'].T + hd['mu_topo_b'],
        vin @ hd['mu_vin_w'].T + hd['mu_vin_b'],
        vout @ hd['mu_vout_w'].T + hd['mu_vout_b'],
        gnd @ hd['mu_gnd_w'].T + hd['mu_gnd_b']], axis=-1)
    logvar = jnp.concatenate([
        cat @ hd['lv_topo_w'].T + hd['lv_topo_b'],
        vin @ hd['lv_vin_w'].T + hd['lv_vin_b'],
        vout @ hd['lv_vout_w'].T + hd['lv_vout_b'],
        gnd @ hd['lv_gnd_w'].T + hd['lv_gnd_b']], axis=-1)
    mu = pl.pallas_call(
        _id_body, out_shape=jax.ShapeDtypeStruct(mu.shape, mu.dtype))(mu)
    z = mu
    return z, mu, logvar


# speedup vs baseline: 3.6859x; 3.5809x over previous
import functools

import jax
import jax.numpy as jnp
from jax import lax
from jax.experimental import pallas as pl
from jax.experimental.pallas import tpu as pltpu
from jax.experimental.pallas import tpu_sc as plsc

_N = 100000
_E = 1600000
_B = 512
_H = 32
_NF = 4
_BK = 2000
_GRID = _N // _BK


def _full_spec(shape):
    return pl.BlockSpec(shape, lambda i: tuple(0 for _ in shape))


def _layer_body(mode, *refs):
    if mode == 0:
        (x_ref, b_ref, w1_ref, b1_ref, w2_ref, p_ref, k_ref, w_ref) = refs
        i = pl.program_id(0)
        xb = x_ref[...]
        r = jnp.maximum(
            jnp.dot(xb, w1_ref[...], preferred_element_type=jnp.float32)
            + b1_ref[...], 0.0)
        p2 = jnp.dot(r, w2_ref[...], preferred_element_type=jnp.float32)
        p_ref[...] = jnp.stack(
            [p2[:, 16 * sl:16 * sl + 16] for sl in range(6)], axis=0)
        t = jnp.zeros((_BK,), jnp.int32)
        m = xb[:, 0]
        for c2 in range(1, _NF):
            xc = xb[:, c2]
            cond = xc > m
            t = jnp.where(cond, jnp.int32(c2), t)
            m = jnp.maximum(m, xc)
        bv = b_ref[...][:, 0]

        @pl.when(i == 0)
        def _():
            w_ref[...] = jnp.full((16, 128), -1, jnp.int32)

        nid = i * _BK + lax.broadcasted_iota(jnp.int32, (_BK, 1), 0)
        for bc in range(4):
            bcol = 128 * bc + lax.broadcasted_iota(jnp.int32, (1, 128), 1)
            mb = bv[:, None] == bcol
            for tt in range(4):
                mt = mb & (t[:, None] == tt)
                cand = jnp.max(jnp.where(mt, nid, -1), axis=0)
                row = 4 * tt + bc
                w_ref[row] = jnp.maximum(w_ref[row], cand)
        k_ref[...] = jnp.maximum(w_ref[...], -1)
        return

    if mode == 1:
        (agg_ref, hp_ref, bias_ref, g_ref, be_ref, pw_ref, pb_ref,
         w1_ref, b1_ref, w2_ref, h_ref, p_ref) = refs
    elif mode == 2:
        (agg_ref, hp_ref, bias_ref, g_ref, be_ref,
         w1_ref, b1_ref, w2_ref, h_ref, p_ref) = refs
    else:
        (agg_ref, hp_ref, bias_ref, g_ref, be_ref, h_ref) = refs

    out = jnp.concatenate([agg_ref[0], agg_ref[1]], axis=-1) + bias_ref[...]
    mean = jnp.mean(out, axis=-1, keepdims=True)
    var = jnp.mean((out - mean) ** 2, axis=-1, keepdims=True)
    out = (out - mean) * lax.rsqrt(var + 1e-5) * g_ref[...] + be_ref[...]
    hp = hp_ref[...]
    if mode == 1:
        res = jnp.dot(hp, pw_ref[...], preferred_element_type=jnp.float32) \
            + pb_ref[...]
    else:
        res = hp
    h = jnp.maximum(out, 0.0) + res
    h_ref[...] = h
    if mode == 3:
        return
    r = jnp.maximum(
        jnp.dot(h, w1_ref[...], preferred_element_type=jnp.float32)
        + b1_ref[...], 0.0)
    p2 = jnp.dot(r, w2_ref[...], preferred_element_type=jnp.float32)
    p_ref[...] = jnp.stack(
        [p2[:, 16 * sl:16 * sl + 16] for sl in range(6)], axis=0)


def _layer0_call(x, batch2d, w1t, b1, w2):
    return pl.pallas_call(
        functools.partial(_layer_body, 0),
        grid=(_GRID,),
        in_specs=[
            pl.BlockSpec((_BK, _NF), lambda i: (i, 0)),
            pl.BlockSpec((_BK, 1), lambda i: (i, 0)),
            _full_spec((_NF, 96)),
            _full_spec((1, 96)),
            _full_spec((96, 96)),
        ],
        out_specs=[
            pl.BlockSpec((6, _BK, 16), lambda i: (0, i, 0)),
            pl.BlockSpec((16, 128), lambda i: (0, 0)),
        ],
        out_shape=[
            jax.ShapeDtypeStruct((6, _N, 16), jnp.float32),
            jax.ShapeDtypeStruct((16, 128), jnp.int32),
        ],
        scratch_shapes=[pltpu.VMEM((16, 128), jnp.int32)],
    )(x, batch2d, w1t, b1, w2)


def _layer_mid_call(mode, agg, hp, bias, lng, lnb, pw, pb, w1t, b1, w2):
    hp_dim = hp.shape[1]
    in_specs = [
        pl.BlockSpec((2, _BK, 16), lambda i: (0, i, 0)),
        pl.BlockSpec((_BK, hp_dim), lambda i: (i, 0)),
        _full_spec((1, 32)),
        _full_spec((1, 32)),
        _full_spec((1, 32)),
    ]
    args = [agg, hp, bias, lng, lnb]
    if mode == 1:
        in_specs += [_full_spec((_NF, 32)), _full_spec((1, 32))]
        args += [pw, pb]
    in_specs += [_full_spec((32, 96)), _full_spec((1, 96)),
                 _full_spec((96, 96))]
    args += [w1t, b1, w2]
    return pl.pallas_call(
        functools.partial(_layer_body, mode),
        grid=(_GRID,),
        in_specs=in_specs,
        out_specs=[
            pl.BlockSpec((_BK, 32), lambda i: (i, 0)),
            pl.BlockSpec((6, _BK, 16), lambda i: (0, i, 0)),
        ],
        out_shape=[
            jax.ShapeDtypeStruct((_N, 32), jnp.float32),
            jax.ShapeDtypeStruct((6, _N, 16), jnp.float32),
        ],
    )(*args)


def _layer_final_call(agg, hp, bias, lng, lnb):
    return pl.pallas_call(
        functools.partial(_layer_body, 3),
        grid=(_GRID,),
        in_specs=[
            pl.BlockSpec((2, _BK, 16), lambda i: (0, i, 0)),
            pl.BlockSpec((_BK, 32), lambda i: (i, 0)),
            _full_spec((1, 32)),
            _full_spec((1, 32)),
            _full_spec((1, 32)),
        ],
        out_specs=pl.BlockSpec((_BK, 32), lambda i: (i, 0)),
        out_shape=jax.ShapeDtypeStruct((_N, 32), jnp.float32),
    )(agg, hp, bias, lng, lnb)


def _boxcox_body(v_ref, o_ref):
    v = v_ref[...]
    o_ref[...] = (jnp.sqrt(1.0 + v) - 1.0) * 2.0


def _boxcox_call(ea):
    eat = ea.T.reshape(3, 12500, 128)
    out = pl.pallas_call(
        _boxcox_body,
        grid=(3,),
        in_specs=[pl.BlockSpec((1, 12500, 128), lambda i: (i, 0, 0))],
        out_specs=pl.BlockSpec((1, 12500, 128), lambda i: (i, 0, 0)),
        out_shape=jax.ShapeDtypeStruct((3, 12500, 128), jnp.float32),
    )(eat)
    return [out[i].reshape(_E) for i in range(3)]


def _heads_body(t_ref, wm_ref, bm_ref, wl_ref, bl_ref, mu_ref, lv_ref):
    t = t_ref[...]
    mu_ref[...] = jnp.dot(t, wm_ref[...],
                          preferred_element_type=jnp.float32) + bm_ref[...]
    lv_ref[...] = jnp.dot(t, wl_ref[...],
                          preferred_element_type=jnp.float32) + bl_ref[...]


def _heads_call(term, wm, bm, wl, bl):
    return pl.pallas_call(
        _heads_body,
        out_shape=[jax.ShapeDtypeStruct((_B, 5), jnp.float32)] * 2,
    )(term, wm, bm, wl, bl)


def _fold_weights(params):
    layers = []
    for lyr in params['layers']:
        lin = lyr['lin_node']
        w1s, b1s, w2lo, w2hi = [], [], [], []
        for nm in ('phi_G', 'phi_C', 'phi_L'):
            p = lyr[nm]
            w1s.append(p['w1'] @ lin)
            b1s.append(p['b1'])
            w2t = p['w2'].T
            w2lo.append(w2t[:, :16])
            w2hi.append(w2t[:, 16:])
        w1t = jnp.concatenate(w1s, axis=0).T
        b1 = jnp.concatenate(b1s)[None, :]
        z = jnp.zeros((32, 32), jnp.float32)
        wt = [jnp.concatenate([lo, hi], axis=1) for lo, hi in zip(w2lo, w2hi)]
        w2 = jnp.concatenate([
            jnp.concatenate([wt[0], z, z], axis=1),
            jnp.concatenate([z, wt[1], z], axis=1),
            jnp.concatenate([z, z, wt[2]], axis=1)], axis=0)
        layers.append({
            'w1t': w1t, 'b1': b1, 'w2': w2,
            'bias': lyr['bias'][None, :], 'lng': lyr['ln_g'][None, :],
            'lnb': lyr['ln_b'][None, :]})
    hd = params['heads']
    wm = jnp.zeros((128, 5), jnp.float32)
    wl = jnp.zeros((128, 5), jnp.float32)
    tw = hd['mu_topo_w']
    wm = wm.at[32:64, 0:2].set(tw[:, 0:32].T)
    wm = wm.at[64:96, 0:2].set(tw[:, 32:64].T)
    wm = wm.at[0:32, 0:2].set(tw[:, 64:96].T)
    wm = wm.at[32:64, 2].set(hd['mu_vin_w'][0])
    wm = wm.at[64:96, 3].set(hd['mu_vout_w'][0])
    wm = wm.at[0:32, 4].set(hd['mu_gnd_w'][0])
    bm = jnp.concatenate([hd['mu_topo_b'], hd['mu_vin_b'],
                          hd['mu_vout_b'], hd['mu_gnd_b']])[None, :]
    tl = hd['lv_topo_w']
    wl = wl.at[32:64, 0:2].set(tl[:, 0:32].T)
    wl = wl.at[64:96, 0:2].set(tl[:, 32:64].T)
    wl = wl.at[0:32, 0:2].set(tl[:, 64:96].T)
    wl = wl.at[32:64, 2].set(hd['lv_vin_w'][0])
    wl = wl.at[64:96, 3].set(hd['lv_vout_w'][0])
    wl = wl.at[0:32, 4].set(hd['lv_gnd_w'][0])
    bl = jnp.concatenate([hd['lv_topo_b'], hd['lv_vin_b'],
                          hd['lv_vout_b'], hd['lv_gnd_b']])[None, :]
    return layers, (wm, bm, wl, bl)


_SUP = 400
_SUB = 80
_NSUB = _SUP // _SUB
_EPT = _E // 16
_NSUPER = _EPT // _SUP
_NP = 100096
_AROWS = _NP // 16
_KPAD = 102400


def _edge_body(pf16, src1, dst1, g1, c1, l1, zsrc, out,
               srcf, srcoff3, dstv, gv, cv, lv, rows16, msg, acc,
               gsem, dsem):
    c = lax.axis_index("c")
    s = lax.axis_index("s")
    r0 = s * _AROWS
    pltpu.sync_copy(zsrc, acc.at[pl.ds(r0, _AROWS), :])
    plsc.subcore_barrier()

    def super_body(sidx, _):
        e0 = s * _EPT + sidx * _SUP
        dcps = [pltpu.make_async_copy(
            dst1.at[pl.ds(e0 + _SUB * j, _SUB)], dstv.at[j], dsem)
            for j in range(_NSUB)]
        for cp in dcps:
            cp.start()
        pltpu.sync_copy(src1.at[pl.ds(e0, _SUP)], srcf)
        pltpu.sync_copy(g1.at[pl.ds(e0, _SUP)], gv)
        pltpu.sync_copy(c1.at[pl.ds(e0, _SUP)], cv)
        pltpu.sync_copy(l1.at[pl.ds(e0, _SUP)], lv)
        coff = c * _N

        def off_body(k, _):
            sv = srcf[pl.ds(16 * k, 16)] + coff
            srcoff3[pl.ds(16 * k, 16)] = sv
            srcoff3[pl.ds(_SUP + 16 * k, 16)] = sv + 2 * _N
            srcoff3[pl.ds(2 * _SUP + 16 * k, 16)] = sv + 4 * _N
            return 0
        lax.fori_loop(0, _SUP // 16, off_body, 0)

        cps = [pltpu.make_async_copy(
            pf16.at[srcoff3.at[pl.ds(_SUB * j, _SUB)]],
            rows16.at[pl.ds(_SUB * j, _SUB), :], gsem)
            for j in range(3 * _NSUB)]
        for cp in cps:
            cp.start()
        for cp in cps:
            cp.wait()

        def grp_body(grp, _):
            wvg = gv[pl.ds(16 * grp, 16)]
            wvc = cv[pl.ds(16 * grp, 16)]
            wvl = lv[pl.ds(16 * grp, 16)]
            for j in range(16):
                e = 16 * grp + j
                rg = rows16[e, pl.ds(0, 16)]
                rc = rows16[_SUP + e, pl.ds(0, 16)]
                rl = rows16[2 * _SUP + e, pl.ds(0, 16)]
                v = (jnp.broadcast_to(wvg[j], (16,)) * rg
                     + jnp.broadcast_to(wvc[j], (16,)) * rc
                     + jnp.broadcast_to(wvl[j], (16,)) * rl)
                msg[e, pl.ds(0, 16)] = v
            return 0
        lax.fori_loop(0, _SUP // 16, grp_body, 0)

        for cp in dcps:
            cp.wait()
        for j in range(_NSUB):
            pltpu.sync_copy(msg.at[pl.ds(_SUB * j, _SUB), :],
                            acc.at[dstv.at[j]], add=True)
        return 0
    lax.fori_loop(0, _NSUPER, super_body, 0)
    plsc.subcore_barrier()
    pltpu.sync_copy(acc.at[pl.ds(r0, _AROWS), :],
                    out.at[c, pl.ds(r0, _AROWS), :])


def _edge_agg_sc(p6, src, dst, g, c, l):
    pf16 = p6.reshape(6 * _N, 16)
    fn = pl.kernel(
        _edge_body,
        out_type=jax.ShapeDtypeStruct((2, _NP, 16), jnp.float32),
        mesh=plsc.VectorSubcoreMesh(core_axis_name="c", subcore_axis_name="s"),
        compiler_params=pltpu.CompilerParams(use_tc_tiling_on_sc=False),
        scratch_types=[
            pltpu.VMEM((_SUP,), jnp.int32),
            pltpu.VMEM((3 * _SUP,), jnp.int32),
            pltpu.VMEM((_NSUB, _SUB), jnp.int32),
            pltpu.VMEM((_SUP,), jnp.float32),
            pltpu.VMEM((_SUP,), jnp.float32),
            pltpu.VMEM((_SUP,), jnp.float32),
            pltpu.VMEM((3 * _SUP, 16), jnp.float32),
            pltpu.VMEM((_SUP, 16), jnp.float32),
            pltpu.VMEM_SHARED((_NP, 16), jnp.float32),
            pltpu.SemaphoreType.DMA,
            pltpu.SemaphoreType.DMA,
        ],
    )
    zsrc = jnp.zeros((_AROWS, 16), jnp.float32)
    return fn(pf16, src, dst, g, c, l, zsrc)


def _t2_body(winf, h3, term, wraw, idxb, rows, gsem):
    c = lax.axis_index("c")
    s = lax.axis_index("s")
    w = c * 16 + s
    k0 = w * 64
    pltpu.sync_copy(winf.at[pl.ds(k0, 64)], wraw)
    for k in range(4):
        sl = pl.ds(16 * k, 16)
        idxb[sl] = jnp.maximum(wraw[sl], 0)
    gcp = pltpu.make_async_copy(h3.at[idxb], rows, gsem)
    gcp.start()
    gcp.wait()
    for k in range(4):
        mf = jnp.where(wraw[pl.ds(16 * k, 16)] < 0, 0.0, 1.0)
        for j2 in range(16):
            j = 16 * k + j2
            sm = jnp.broadcast_to(mf[j2], (16,))
            rows[j, pl.ds(0, 16)] = rows[j, pl.ds(0, 16)] * sm
            rows[j, pl.ds(16, 16)] = rows[j, pl.ds(16, 16)] * sm
    pltpu.sync_copy(rows, term.at[pl.ds(k0, 64), :])


def _terminal_sc(wtab, h3):
    winf = wtab.reshape(2048)
    term = pl.kernel(
        _t2_body,
        out_type=jax.ShapeDtypeStruct((2048, 32), jnp.float32),
        mesh=plsc.VectorSubcoreMesh(core_axis_name="c", subcore_axis_name="s"),
        compiler_params=pltpu.CompilerParams(use_tc_tiling_on_sc=False),
        scratch_types=[
            pltpu.VMEM((64,), jnp.int32),
            pltpu.VMEM((64,), jnp.int32),
            pltpu.VMEM((64, 32), jnp.float32),
            pltpu.SemaphoreType.DMA,
        ],
    )(winf, h3)
    return term.reshape(4, _B, 32).transpose(1, 0, 2).reshape(_B, 128)


def _edge_agg_jnp(p3, src, dst, g, c, l):
    pg = jnp.concatenate([p3[0, :, 0:16], p3[1, :, 0:16]], axis=1)
    pc = jnp.concatenate([p3[0, :, 16:32], p3[1, :, 16:32]], axis=1)
    plt = jnp.concatenate([p3[0, :, 32:48], p3[1, :, 32:48]], axis=1)
    msg = (g[:, None] * pg[src] + c[:, None] * pc[src] + l[:, None] * plt[src])
    agg = jnp.zeros((_N, 32), jnp.float32).at[dst].add(msg)
    return jnp.stack([agg[:, :16], agg[:, 16:]])


def _terminal_jnp(keys, h):
    nid = jnp.arange(_N, dtype=jnp.int32)
    win = jnp.full((4 * _B,), -1, jnp.int32).at[keys[:, 0]].max(nid)
    term = jnp.where(win[:, None] >= 0, h[jnp.maximum(win, 0)], 0.0)
    return term.reshape(_B, 128)


def kernel(x, edge_attr, params, edge_index, batch):
    src = edge_index[0].astype(jnp.int32)
    dst = edge_index[1].astype(jnp.int32)
    layers, heads_w = _fold_weights(params)
    batch2d = batch.astype(jnp.int32).reshape(_N, 1)

    g, c, l = _boxcox_call(edge_attr)

    p3, wtab = _layer0_call(x, batch2d, layers[0]['w1t'], layers[0]['b1'],
                            layers[0]['w2'])
    agg = _edge_agg_sc(p3, src, dst, g, c, l)

    pw = params['proj']['w'].T
    pb = params['proj']['b'][None, :]
    h1, p3 = _layer_mid_call(1, agg, x, layers[0]['bias'], layers[0]['lng'],
                             layers[0]['lnb'], pw, pb, layers[1]['w1t'],
                             layers[1]['b1'], layers[1]['w2'])
    agg = _edge_agg_sc(p3, src, dst, g, c, l)

    h2, p3 = _layer_mid_call(2, agg, h1, layers[1]['bias'], layers[1]['lng'],
                             layers[1]['lnb'], None, None, layers[2]['w1t'],
                             layers[2]['b1'], layers[2]['w2'])
    agg = _edge_agg_sc(p3, src, dst, g, c, l)

    h3 = _layer_final_call(agg, h2, layers[2]['bias'], layers[2]['lng'],
                           layers[2]['lnb'])

    term = _terminal_sc(wtab, h3)
    mu, logvar = _heads_call(term, *heads_w)
    return mu, mu, logvar


# speedup vs baseline: 4.4180x; 1.1986x over previous
import functools

import jax
import jax.numpy as jnp
from jax import lax
from jax.experimental import pallas as pl
from jax.experimental.pallas import tpu as pltpu
from jax.experimental.pallas import tpu_sc as plsc

_N = 100000
_E = 1600000
_B = 512
_H = 32
_NF = 4
_BK = 2000
_GRID = _N // _BK


def _full_spec(shape):
    return pl.BlockSpec(shape, lambda i: tuple(0 for _ in shape))


def _layer_body(mode, *refs):
    if mode == 0:
        (x_ref, b_ref, w1_ref, b1_ref, w2_ref, p_ref, k_ref, w_ref) = refs
        i = pl.program_id(0)
        xb = x_ref[...]
        r = jnp.maximum(
            jnp.dot(xb, w1_ref[...], preferred_element_type=jnp.float32)
            + b1_ref[...], 0.0)
        p2 = jnp.dot(r, w2_ref[...], preferred_element_type=jnp.float32)
        p_ref[...] = jnp.stack(
            [p2[:, 16 * sl:16 * sl + 16] for sl in range(6)], axis=0)
        t = jnp.zeros((_BK,), jnp.int32)
        m = xb[:, 0]
        for c2 in range(1, _NF):
            xc = xb[:, c2]
            cond = xc > m
            t = jnp.where(cond, jnp.int32(c2), t)
            m = jnp.maximum(m, xc)
        bv = b_ref[...][:, 0]

        @pl.when(i == 0)
        def _():
            w_ref[...] = jnp.full((16, 128), -1, jnp.int32)

        nid = i * _BK + lax.broadcasted_iota(jnp.int32, (_BK, 1), 0)
        for bc in range(4):
            bcol = 128 * bc + lax.broadcasted_iota(jnp.int32, (1, 128), 1)
            mb = bv[:, None] == bcol
            for tt in range(4):
                mt = mb & (t[:, None] == tt)
                cand = jnp.max(jnp.where(mt, nid, -1), axis=0)
                row = 4 * tt + bc
                w_ref[row] = jnp.maximum(w_ref[row], cand)
        k_ref[...] = jnp.maximum(w_ref[...], -1)
        return

    if mode == 1:
        (agg_ref, hp_ref, bias_ref, g_ref, be_ref, pw_ref, pb_ref,
         w1_ref, b1_ref, w2_ref, h_ref, p_ref) = refs
    elif mode == 2:
        (agg_ref, hp_ref, bias_ref, g_ref, be_ref,
         w1_ref, b1_ref, w2_ref, h_ref, p_ref) = refs
    else:
        (agg_ref, hp_ref, bias_ref, g_ref, be_ref, h_ref) = refs

    out = jnp.concatenate([agg_ref[0], agg_ref[1]], axis=-1) + bias_ref[...]
    mean = jnp.mean(out, axis=-1, keepdims=True)
    var = jnp.mean((out - mean) ** 2, axis=-1, keepdims=True)
    out = (out - mean) * lax.rsqrt(var + 1e-5) * g_ref[...] + be_ref[...]
    hp = hp_ref[...]
    if mode == 1:
        res = jnp.dot(hp, pw_ref[...], preferred_element_type=jnp.float32) \
            + pb_ref[...]
    else:
        res = hp
    h = jnp.maximum(out, 0.0) + res
    h_ref[...] = h
    if mode == 3:
        return
    r = jnp.maximum(
        jnp.dot(h, w1_ref[...], preferred_element_type=jnp.float32)
        + b1_ref[...], 0.0)
    p2 = jnp.dot(r, w2_ref[...], preferred_element_type=jnp.float32)
    p_ref[...] = jnp.stack(
        [p2[:, 16 * sl:16 * sl + 16] for sl in range(6)], axis=0)


def _layer0_call(x, batch2d, w1t, b1, w2):
    return pl.pallas_call(
        functools.partial(_layer_body, 0),
        grid=(_GRID,),
        in_specs=[
            pl.BlockSpec((_BK, _NF), lambda i: (i, 0)),
            pl.BlockSpec((_BK, 1), lambda i: (i, 0)),
            _full_spec((_NF, 96)),
            _full_spec((1, 96)),
            _full_spec((96, 96)),
        ],
        out_specs=[
            pl.BlockSpec((6, _BK, 16), lambda i: (0, i, 0)),
            pl.BlockSpec((16, 128), lambda i: (0, 0)),
        ],
        out_shape=[
            jax.ShapeDtypeStruct((6, _N, 16), jnp.float32),
            jax.ShapeDtypeStruct((16, 128), jnp.int32),
        ],
        scratch_shapes=[pltpu.VMEM((16, 128), jnp.int32)],
    )(x, batch2d, w1t, b1, w2)


def _layer_mid_call(mode, agg, hp, bias, lng, lnb, pw, pb, w1t, b1, w2):
    hp_dim = hp.shape[1]
    in_specs = [
        pl.BlockSpec((2, _BK, 16), lambda i: (0, i, 0)),
        pl.BlockSpec((_BK, hp_dim), lambda i: (i, 0)),
        _full_spec((1, 32)),
        _full_spec((1, 32)),
        _full_spec((1, 32)),
    ]
    args = [agg, hp, bias, lng, lnb]
    if mode == 1:
        in_specs += [_full_spec((_NF, 32)), _full_spec((1, 32))]
        args += [pw, pb]
    in_specs += [_full_spec((32, 96)), _full_spec((1, 96)),
                 _full_spec((96, 96))]
    args += [w1t, b1, w2]
    return pl.pallas_call(
        functools.partial(_layer_body, mode),
        grid=(_GRID,),
        in_specs=in_specs,
        out_specs=[
            pl.BlockSpec((_BK, 32), lambda i: (i, 0)),
            pl.BlockSpec((6, _BK, 16), lambda i: (0, i, 0)),
        ],
        out_shape=[
            jax.ShapeDtypeStruct((_N, 32), jnp.float32),
            jax.ShapeDtypeStruct((6, _N, 16), jnp.float32),
        ],
    )(*args)


def _layer_final_call(agg, hp, bias, lng, lnb):
    return pl.pallas_call(
        functools.partial(_layer_body, 3),
        grid=(_GRID,),
        in_specs=[
            pl.BlockSpec((2, _BK, 16), lambda i: (0, i, 0)),
            pl.BlockSpec((_BK, 32), lambda i: (i, 0)),
            _full_spec((1, 32)),
            _full_spec((1, 32)),
            _full_spec((1, 32)),
        ],
        out_specs=pl.BlockSpec((_BK, 32), lambda i: (i, 0)),
        out_shape=jax.ShapeDtypeStruct((_N, 32), jnp.float32),
    )(agg, hp, bias, lng, lnb)


def _boxcox_body(v_ref, o_ref):
    v = v_ref[...]
    o_ref[...] = (jnp.sqrt(1.0 + v) - 1.0) * 2.0


def _boxcox_call(ea):
    eat = ea.T.reshape(3, 12500, 128)
    out = pl.pallas_call(
        _boxcox_body,
        grid=(3,),
        in_specs=[pl.BlockSpec((1, 12500, 128), lambda i: (i, 0, 0))],
        out_specs=pl.BlockSpec((1, 12500, 128), lambda i: (i, 0, 0)),
        out_shape=jax.ShapeDtypeStruct((3, 12500, 128), jnp.float32),
    )(eat)
    return [out[i].reshape(_E) for i in range(3)]


def _heads_body(t_ref, wm_ref, bm_ref, wl_ref, bl_ref, mu_ref, lv_ref):
    t = t_ref[...]
    mu_ref[...] = jnp.dot(t, wm_ref[...],
                          preferred_element_type=jnp.float32) + bm_ref[...]
    lv_ref[...] = jnp.dot(t, wl_ref[...],
                          preferred_element_type=jnp.float32) + bl_ref[...]


def _heads_call(term, wm, bm, wl, bl):
    return pl.pallas_call(
        _heads_body,
        out_shape=[jax.ShapeDtypeStruct((_B, 5), jnp.float32)] * 2,
    )(term, wm, bm, wl, bl)


def _fold_weights(params):
    layers = []
    for lyr in params['layers']:
        lin = lyr['lin_node']
        w1s, b1s, w2lo, w2hi = [], [], [], []
        for nm in ('phi_G', 'phi_C', 'phi_L'):
            p = lyr[nm]
            w1s.append(p['w1'] @ lin)
            b1s.append(p['b1'])
            w2t = p['w2'].T
            w2lo.append(w2t[:, :16])
            w2hi.append(w2t[:, 16:])
        w1t = jnp.concatenate(w1s, axis=0).T
        b1 = jnp.concatenate(b1s)[None, :]
        z = jnp.zeros((32, 32), jnp.float32)
        wt = [jnp.concatenate([lo, hi], axis=1) for lo, hi in zip(w2lo, w2hi)]
        w2 = jnp.concatenate([
            jnp.concatenate([wt[0], z, z], axis=1),
            jnp.concatenate([z, wt[1], z], axis=1),
            jnp.concatenate([z, z, wt[2]], axis=1)], axis=0)
        layers.append({
            'w1t': w1t, 'b1': b1, 'w2': w2,
            'bias': lyr['bias'][None, :], 'lng': lyr['ln_g'][None, :],
            'lnb': lyr['ln_b'][None, :]})
    hd = params['heads']
    wm = jnp.zeros((128, 5), jnp.float32)
    wl = jnp.zeros((128, 5), jnp.float32)
    tw = hd['mu_topo_w']
    wm = wm.at[32:64, 0:2].set(tw[:, 0:32].T)
    wm = wm.at[64:96, 0:2].set(tw[:, 32:64].T)
    wm = wm.at[0:32, 0:2].set(tw[:, 64:96].T)
    wm = wm.at[32:64, 2].set(hd['mu_vin_w'][0])
    wm = wm.at[64:96, 3].set(hd['mu_vout_w'][0])
    wm = wm.at[0:32, 4].set(hd['mu_gnd_w'][0])
    bm = jnp.concatenate([hd['mu_topo_b'], hd['mu_vin_b'],
                          hd['mu_vout_b'], hd['mu_gnd_b']])[None, :]
    tl = hd['lv_topo_w']
    wl = wl.at[32:64, 0:2].set(tl[:, 0:32].T)
    wl = wl.at[64:96, 0:2].set(tl[:, 32:64].T)
    wl = wl.at[0:32, 0:2].set(tl[:, 64:96].T)
    wl = wl.at[32:64, 2].set(hd['lv_vin_w'][0])
    wl = wl.at[64:96, 3].set(hd['lv_vout_w'][0])
    wl = wl.at[0:32, 4].set(hd['lv_gnd_w'][0])
    bl = jnp.concatenate([hd['lv_topo_b'], hd['lv_vin_b'],
                          hd['lv_vout_b'], hd['lv_gnd_b']])[None, :]
    return layers, (wm, bm, wl, bl)


_SUP = 400
_SUB = 80
_NSUB = _SUP // _SUB
_EPT = _E // 16
_NSUPER = _EPT // _SUP
_NP = 100096
_AROWS = _NP // 16
_KPAD = 102400


def _edge_body(pf16, src1, dst1, g1, c1, l1, zsrc, out,
               srcf, srcoff3, dstv, gv, cv, lv, rows16, msg, acc,
               gsem, dsem, lsem, ssem):
    c = lax.axis_index("c")
    s = lax.axis_index("s")
    r0 = s * _AROWS
    pltpu.sync_copy(zsrc, acc.at[pl.ds(r0, _AROWS), :])
    plsc.subcore_barrier()

    def super_body(sidx, _):
        e0 = s * _EPT + sidx * _SUP
        dcps = [pltpu.make_async_copy(
            dst1.at[pl.ds(e0 + _SUB * j, _SUB)], dstv.at[j], dsem)
            for j in range(_NSUB)]
        for cp in dcps:
            cp.start()
        lcps = [pltpu.make_async_copy(a.at[pl.ds(e0, _SUP)], b, lsem)
                for a, b in ((src1, srcf), (g1, gv), (c1, cv), (l1, lv))]
        for cp in lcps:
            cp.start()
        for cp in lcps:
            cp.wait()
        coff = c * _N

        def off_body(k, _):
            sv = srcf[pl.ds(16 * k, 16)] + coff
            srcoff3[pl.ds(16 * k, 16)] = sv
            srcoff3[pl.ds(_SUP + 16 * k, 16)] = sv + 2 * _N
            srcoff3[pl.ds(2 * _SUP + 16 * k, 16)] = sv + 4 * _N
            return 0
        lax.fori_loop(0, _SUP // 16, off_body, 0)

        cps = [pltpu.make_async_copy(
            pf16.at[srcoff3.at[pl.ds(_SUB * j, _SUB)]],
            rows16.at[pl.ds(_SUB * j, _SUB), :], gsem)
            for j in range(3 * _NSUB)]
        for cp in cps:
            cp.start()
        for cp in cps:
            cp.wait()

        def grp_body(grp, _):
            wvg = gv[pl.ds(16 * grp, 16)]
            wvc = cv[pl.ds(16 * grp, 16)]
            wvl = lv[pl.ds(16 * grp, 16)]
            for j in range(16):
                e = 16 * grp + j
                rg = rows16[e, pl.ds(0, 16)]
                rc = rows16[_SUP + e, pl.ds(0, 16)]
                rl = rows16[2 * _SUP + e, pl.ds(0, 16)]
                v = (jnp.broadcast_to(wvg[j], (16,)) * rg
                     + jnp.broadcast_to(wvc[j], (16,)) * rc
                     + jnp.broadcast_to(wvl[j], (16,)) * rl)
                msg[e, pl.ds(0, 16)] = v
            return 0
        lax.fori_loop(0, _SUP // 16, grp_body, 0)

        for cp in dcps:
            cp.wait()
        scps = [pltpu.async_copy(
            msg.at[pl.ds(_SUB * j, _SUB), :],
            acc.at[dstv.at[j]], ssem, add=True) for j in range(_NSUB)]
        for cp in scps:
            cp.wait()
        return 0
    lax.fori_loop(0, _NSUPER, super_body, 0)
    plsc.subcore_barrier()
    pltpu.sync_copy(acc.at[pl.ds(r0, _AROWS), :],
                    out.at[c, pl.ds(r0, _AROWS), :])


def _edge_agg_sc(p6, src, dst, g, c, l):
    pf16 = p6.reshape(6 * _N, 16)
    fn = pl.kernel(
        _edge_body,
        out_type=jax.ShapeDtypeStruct((2, _NP, 16), jnp.float32),
        mesh=plsc.VectorSubcoreMesh(core_axis_name="c", subcore_axis_name="s"),
        compiler_params=pltpu.CompilerParams(use_tc_tiling_on_sc=False),
        scratch_types=[
            pltpu.VMEM((_SUP,), jnp.int32),
            pltpu.VMEM((3 * _SUP,), jnp.int32),
            pltpu.VMEM((_NSUB, _SUB), jnp.int32),
            pltpu.VMEM((_SUP,), jnp.float32),
            pltpu.VMEM((_SUP,), jnp.float32),
            pltpu.VMEM((_SUP,), jnp.float32),
            pltpu.VMEM((3 * _SUP, 16), jnp.float32),
            pltpu.VMEM((_SUP, 16), jnp.float32),
            pltpu.VMEM_SHARED((_NP, 16), jnp.float32),
            pltpu.SemaphoreType.DMA,
            pltpu.SemaphoreType.DMA,
            pltpu.SemaphoreType.DMA,
            pltpu.SemaphoreType.DMA,
        ],
    )
    zsrc = jnp.zeros((_AROWS, 16), jnp.float32)
    return fn(pf16, src, dst, g, c, l, zsrc)


def _t2_body(winf, h3, term, wraw, idxb, rows, gsem):
    c = lax.axis_index("c")
    s = lax.axis_index("s")
    w = c * 16 + s
    k0 = w * 64
    pltpu.sync_copy(winf.at[pl.ds(k0, 64)], wraw)
    for k in range(4):
        sl = pl.ds(16 * k, 16)
        idxb[sl] = jnp.maximum(wraw[sl], 0)
    gcp = pltpu.make_async_copy(h3.at[idxb], rows, gsem)
    gcp.start()
    gcp.wait()
    for k in range(4):
        mf = jnp.where(wraw[pl.ds(16 * k, 16)] < 0, 0.0, 1.0)
        for j2 in range(16):
            j = 16 * k + j2
            sm = jnp.broadcast_to(mf[j2], (16,))
            rows[j, pl.ds(0, 16)] = rows[j, pl.ds(0, 16)] * sm
            rows[j, pl.ds(16, 16)] = rows[j, pl.ds(16, 16)] * sm
    pltpu.sync_copy(rows, term.at[pl.ds(k0, 64), :])


def _terminal_sc(wtab, h3):
    winf = wtab.reshape(2048)
    term = pl.kernel(
        _t2_body,
        out_type=jax.ShapeDtypeStruct((2048, 32), jnp.float32),
        mesh=plsc.VectorSubcoreMesh(core_axis_name="c", subcore_axis_name="s"),
        compiler_params=pltpu.CompilerParams(use_tc_tiling_on_sc=False),
        scratch_types=[
            pltpu.VMEM((64,), jnp.int32),
            pltpu.VMEM((64,), jnp.int32),
            pltpu.VMEM((64, 32), jnp.float32),
            pltpu.SemaphoreType.DMA,
        ],
    )(winf, h3)
    return term.reshape(4, _B, 32).transpose(1, 0, 2).reshape(_B, 128)


def _edge_agg_jnp(p3, src, dst, g, c, l):
    pg = jnp.concatenate([p3[0, :, 0:16], p3[1, :, 0:16]], axis=1)
    pc = jnp.concatenate([p3[0, :, 16:32], p3[1, :, 16:32]], axis=1)
    plt = jnp.concatenate([p3[0, :, 32:48], p3[1, :, 32:48]], axis=1)
    msg = (g[:, None] * pg[src] + c[:, None] * pc[src] + l[:, None] * plt[src])
    agg = jnp.zeros((_N, 32), jnp.float32).at[dst].add(msg)
    return jnp.stack([agg[:, :16], agg[:, 16:]])


def _terminal_jnp(keys, h):
    nid = jnp.arange(_N, dtype=jnp.int32)
    win = jnp.full((4 * _B,), -1, jnp.int32).at[keys[:, 0]].max(nid)
    term = jnp.where(win[:, None] >= 0, h[jnp.maximum(win, 0)], 0.0)
    return term.reshape(_B, 128)


def kernel(x, edge_attr, params, edge_index, batch):
    src = edge_index[0].astype(jnp.int32)
    dst = edge_index[1].astype(jnp.int32)
    layers, heads_w = _fold_weights(params)
    batch2d = batch.astype(jnp.int32).reshape(_N, 1)

    g, c, l = _boxcox_call(edge_attr)

    p3, wtab = _layer0_call(x, batch2d, layers[0]['w1t'], layers[0]['b1'],
                            layers[0]['w2'])
    agg = _edge_agg_sc(p3, src, dst, g, c, l)

    pw = params['proj']['w'].T
    pb = params['proj']['b'][None, :]
    h1, p3 = _layer_mid_call(1, agg, x, layers[0]['bias'], layers[0]['lng'],
                             layers[0]['lnb'], pw, pb, layers[1]['w1t'],
                             layers[1]['b1'], layers[1]['w2'])
    agg = _edge_agg_sc(p3, src, dst, g, c, l)

    h2, p3 = _layer_mid_call(2, agg, h1, layers[1]['bias'], layers[1]['lng'],
                             layers[1]['lnb'], None, None, layers[2]['w1t'],
                             layers[2]['b1'], layers[2]['w2'])
    agg = _edge_agg_sc(p3, src, dst, g, c, l)

    h3 = _layer_final_call(agg, h2, layers[2]['bias'], layers[2]['lng'],
                           layers[2]['lnb'])

    term = _terminal_sc(wtab, h3)
    mu, logvar = _heads_call(term, *heads_w)
    return mu, mu, logvar
